# Initial kernel scaffold; baseline (speedup 1.0000x reference)
import jax
import jax.numpy as jnp
from jax.experimental import pallas as pl


def kernel(x_in, emb, W_ih, W_hh, b_ih, b_hh, W_fc, b_fc):
    raise NotImplementedError("write your pallas kernel here")


# speedup vs baseline: 9.9290x; 9.9290x over previous
import functools

import jax
import jax.numpy as jnp
from jax.experimental import pallas as pl
from jax.experimental.pallas import tpu as pltpu


def _gru_kernel(x_ref, emb_ref, W_ihT_ref, W_hhT_ref, b_ih_ref, b_hh_ref,
                W_fcT_ref, b_fc_ref, out_ref, h_ref, G_ref, *, H):
    t = pl.program_id(0)
    B = h_ref.shape[0]
    V = G_ref.shape[0]

    @pl.when(t == 0)
    def _init():
        G_ref[...] = (
            jnp.dot(emb_ref[...], W_ihT_ref[...],
                    preferred_element_type=jnp.float32)
            + b_ih_ref[...]
        )
        h_ref[...] = jnp.zeros_like(h_ref)

    idx = x_ref[t]
    onehot = (idx[:, None]
              == jax.lax.broadcasted_iota(jnp.int32, (B, V), 1)
              ).astype(jnp.float32)
    gi = jnp.dot(onehot, G_ref[...], preferred_element_type=jnp.float32)

    h = h_ref[...]
    gh = (jnp.dot(h, W_hhT_ref[...], preferred_element_type=jnp.float32)
          + b_hh_ref[...])

    i_r, i_z, i_n = gi[:, :H], gi[:, H:2 * H], gi[:, 2 * H:]
    h_r, h_z, h_n = gh[:, :H], gh[:, H:2 * H], gh[:, 2 * H:]
    r = jax.nn.sigmoid(i_r + h_r)
    z = jax.nn.sigmoid(i_z + h_z)
    n = jnp.tanh(i_n + r * h_n)
    h_new = (1.0 - z) * n + z * h
    h_ref[...] = h_new

    out_ref[0] = (
        jnp.dot(h_new, W_fcT_ref[...], preferred_element_type=jnp.float32)
        + b_fc_ref[...]
    )


def kernel(x_in, emb, W_ih, W_hh, b_ih, b_hh, W_fc, b_fc):
    B, S = x_in.shape
    V, E = emb.shape
    H = W_hh.shape[1]

    x = x_in.astype(jnp.int32).T
    W_ihT = W_ih.T
    W_hhT = W_hh.T
    W_fcT = W_fc.T
    b_ih2 = b_ih.reshape(1, -1)
    b_hh2 = b_hh.reshape(1, -1)
    b_fc2 = b_fc.reshape(1, -1)

    full = lambda shape: pl.BlockSpec(shape, lambda t: (0,) * len(shape))
    out = pl.pallas_call(
        functools.partial(_gru_kernel, H=H),
        grid=(S,),
        in_specs=[
            full((S, B)),
            full((V, E)),
            full((E, 3 * H)),
            full((H, 3 * H)),
            full((1, 3 * H)),
            full((1, 3 * H)),
            full((H, V)),
            full((1, V)),
        ],
        out_specs=pl.BlockSpec((1, B, V), lambda t: (t, 0, 0)),
        out_shape=jax.ShapeDtypeStruct((S, B, V), jnp.float32),
        scratch_shapes=[
            pltpu.VMEM((B, H), jnp.float32),
            pltpu.VMEM((V, 3 * H), jnp.float32),
        ],
    )(x, emb, W_ihT, W_hhT, b_ih2, b_hh2, W_fcT, b_fc2)
    return jnp.swapaxes(out, 0, 1)


# speedup vs baseline: 10.5653x; 1.0641x over previous
import functools

import jax
import jax.numpy as jnp
from jax.experimental import pallas as pl
from jax.experimental.pallas import tpu as pltpu


def _gru_kernel(x_ref, emb_ref, W_ihT_ref, W_hhT_ref, b_ih_ref, b_hh_ref,
                W_fcT_ref, b_fc_ref, out_ref, h_ref, G_ref, *, H):
    t = pl.program_id(0)
    B = h_ref.shape[0]
    V = G_ref.shape[0]

    @pl.when(t == 0)
    def _init():
        G_ref[...] = (
            jnp.dot(emb_ref[...], W_ihT_ref[...],
                    preferred_element_type=jnp.float32)
            + b_ih_ref[...]
        ).astype(jnp.bfloat16)
        h_ref[...] = jnp.zeros_like(h_ref)

    idx = x_ref[t]
    onehot = (idx[:, None]
              == jax.lax.broadcasted_iota(jnp.int32, (B, V), 1)
              ).astype(jnp.bfloat16)
    gi = jnp.dot(onehot, G_ref[...], preferred_element_type=jnp.float32)

    h = h_ref[...]
    gh = (jnp.dot(h.astype(jnp.bfloat16), W_hhT_ref[...],
                  preferred_element_type=jnp.float32)
          + b_hh_ref[...])

    i_r, i_z, i_n = gi[:, :H], gi[:, H:2 * H], gi[:, 2 * H:]
    h_r, h_z, h_n = gh[:, :H], gh[:, H:2 * H], gh[:, 2 * H:]
    r = jax.nn.sigmoid(i_r + h_r)
    z = jax.nn.sigmoid(i_z + h_z)
    n = jnp.tanh(i_n + r * h_n)
    h_new = (1.0 - z) * n + z * h
    h_ref[...] = h_new

    out_ref[0] = (
        jnp.dot(h_new.astype(jnp.bfloat16), W_fcT_ref[...],
                preferred_element_type=jnp.float32)
        + b_fc_ref[...]
    )


def kernel(x_in, emb, W_ih, W_hh, b_ih, b_hh, W_fc, b_fc):
    B, S = x_in.shape
    V, E = emb.shape
    H = W_hh.shape[1]

    x = x_in.astype(jnp.int32).T
    W_ihT = W_ih.T
    W_hhT = W_hh.T.astype(jnp.bfloat16)
    W_fcT = W_fc.T.astype(jnp.bfloat16)
    b_ih2 = b_ih.reshape(1, -1)
    b_hh2 = b_hh.reshape(1, -1)
    b_fc2 = b_fc.reshape(1, -1)

    full = lambda shape: pl.BlockSpec(shape, lambda t: (0,) * len(shape))
    out = pl.pallas_call(
        functools.partial(_gru_kernel, H=H),
        grid=(S,),
        in_specs=[
            full((S, B)),
            full((V, E)),
            full((E, 3 * H)),
            full((H, 3 * H)),
            full((1, 3 * H)),
            full((1, 3 * H)),
            full((H, V)),
            full((1, V)),
        ],
        out_specs=pl.BlockSpec((1, B, V), lambda t: (t, 0, 0)),
        out_shape=jax.ShapeDtypeStruct((S, B, V), jnp.float32),
        scratch_shapes=[
            pltpu.VMEM((B, H), jnp.float32),
            pltpu.VMEM((V, 3 * H), jnp.bfloat16),
        ],
    )(x, emb, W_ihT, W_hhT, b_ih2, b_hh2, W_fcT, b_fc2)
    return jnp.swapaxes(out, 0, 1)


# speedup vs baseline: 11.3507x; 1.0743x over previous
import functools

import jax
import jax.numpy as jnp
from jax.experimental import pallas as pl
from jax.experimental.pallas import tpu as pltpu


def _gru_kernel(x_ref, emb_ref, W_ihT_ref, W_hhT_ref, b_ih_ref, b_hh_ref,
                W_fcT_ref, b_fc_ref, out_ref, h_ref, G_ref, *, H, U):
    t = pl.program_id(0)
    B = h_ref.shape[0]
    V = G_ref.shape[0]

    @pl.when(t == 0)
    def _init():
        G_ref[...] = (
            jnp.dot(emb_ref[...], W_ihT_ref[...],
                    preferred_element_type=jnp.float32)
            + b_ih_ref[...]
        ).astype(jnp.bfloat16)
        h_ref[...] = jnp.zeros_like(h_ref)

    h = h_ref[...]
    for u in range(U):
        idx = x_ref[t * U + u]
        onehot = (idx[:, None]
                  == jax.lax.broadcasted_iota(jnp.int32, (B, V), 1)
                  ).astype(jnp.bfloat16)
        gi = jnp.dot(onehot, G_ref[...], preferred_element_type=jnp.float32)

        gh = (jnp.dot(h.astype(jnp.bfloat16), W_hhT_ref[...],
                      preferred_element_type=jnp.float32)
              + b_hh_ref[...])

        i_r, i_z, i_n = gi[:, :H], gi[:, H:2 * H], gi[:, 2 * H:]
        h_r, h_z, h_n = gh[:, :H], gh[:, H:2 * H], gh[:, 2 * H:]
        r = jax.nn.sigmoid(i_r + h_r)
        z = jax.nn.sigmoid(i_z + h_z)
        n = jnp.tanh(i_n + r * h_n)
        h = (1.0 - z) * n + z * h

        out_ref[u] = (
            jnp.dot(h.astype(jnp.bfloat16), W_fcT_ref[...],
                    preferred_element_type=jnp.float32)
            + b_fc_ref[...]
        )
    h_ref[...] = h


def kernel(x_in, emb, W_ih, W_hh, b_ih, b_hh, W_fc, b_fc):
    B, S = x_in.shape
    V, E = emb.shape
    H = W_hh.shape[1]

    x = x_in.astype(jnp.int32).T
    W_ihT = W_ih.T
    W_hhT = W_hh.T.astype(jnp.bfloat16)
    W_fcT = W_fc.T.astype(jnp.bfloat16)
    b_ih2 = b_ih.reshape(1, -1)
    b_hh2 = b_hh.reshape(1, -1)
    b_fc2 = b_fc.reshape(1, -1)

    U = 2
    full = lambda shape: pl.BlockSpec(shape, lambda t: (0,) * len(shape))
    out = pl.pallas_call(
        functools.partial(_gru_kernel, H=H, U=U),
        grid=(S // U,),
        in_specs=[
            full((S, B)),
            full((V, E)),
            full((E, 3 * H)),
            full((H, 3 * H)),
            full((1, 3 * H)),
            full((1, 3 * H)),
            full((H, V)),
            full((1, V)),
        ],
        out_specs=pl.BlockSpec((U, B, V), lambda t: (t, 0, 0)),
        out_shape=jax.ShapeDtypeStruct((S, B, V), jnp.float32),
        scratch_shapes=[
            pltpu.VMEM((B, H), jnp.float32),
            pltpu.VMEM((V, 3 * H), jnp.bfloat16),
        ],
    )(x, emb, W_ihT, W_hhT, b_ih2, b_hh2, W_fcT, b_fc2)
    return jnp.swapaxes(out, 0, 1)


# speedup vs baseline: 11.8447x; 1.0435x over previous
import functools

import jax
import jax.numpy as jnp
from jax.experimental import pallas as pl
from jax.experimental.pallas import tpu as pltpu


def _gru_kernel(x_ref, emb_ref, W_ihT_ref, W_hhT_ref, b_ih_ref, b_hh_ref,
                W_fcT_ref, b_fc_ref, out_ref, h_ref, G_ref, *, H, U):
    t = pl.program_id(0)
    B = h_ref.shape[0]
    V = G_ref.shape[0]

    @pl.when(t == 0)
    def _init():
        G_ref[...] = (
            jnp.dot(emb_ref[...], W_ihT_ref[...],
                    preferred_element_type=jnp.float32)
            + b_ih_ref[...]
        ).astype(jnp.bfloat16)
        h_ref[...] = jnp.zeros_like(h_ref)

    h = h_ref[...]
    for u in range(U):
        idx = x_ref[t * U + u]
        onehot = (idx[:, None]
                  == jax.lax.broadcasted_iota(jnp.int32, (B, V), 1)
                  ).astype(jnp.bfloat16)
        gi = jnp.dot(onehot, G_ref[...], preferred_element_type=jnp.float32)

        gh = (jnp.dot(h.astype(jnp.bfloat16), W_hhT_ref[...],
                      preferred_element_type=jnp.float32)
              + b_hh_ref[...])

        i_r, i_z, i_n = gi[:, :H], gi[:, H:2 * H], gi[:, 2 * H:]
        h_r, h_z, h_n = gh[:, :H], gh[:, H:2 * H], gh[:, 2 * H:]
        r = jax.nn.sigmoid(i_r + h_r)
        z = jax.nn.sigmoid(i_z + h_z)
        n = jnp.tanh(i_n + r * h_n)
        h = (1.0 - z) * n + z * h

        out_ref[u] = (
            jnp.dot(h.astype(jnp.bfloat16), W_fcT_ref[...],
                    preferred_element_type=jnp.float32)
            + b_fc_ref[...]
        )
    h_ref[...] = h


def kernel(x_in, emb, W_ih, W_hh, b_ih, b_hh, W_fc, b_fc):
    B, S = x_in.shape
    V, E = emb.shape
    H = W_hh.shape[1]

    x = x_in.astype(jnp.int32).T
    W_ihT = W_ih.T
    W_hhT = W_hh.T.astype(jnp.bfloat16)
    W_fcT = W_fc.T.astype(jnp.bfloat16)
    b_ih2 = b_ih.reshape(1, -1)
    b_hh2 = b_hh.reshape(1, -1)
    b_fc2 = b_fc.reshape(1, -1)

    U = 4
    full = lambda shape: pl.BlockSpec(shape, lambda t: (0,) * len(shape))
    out = pl.pallas_call(
        functools.partial(_gru_kernel, H=H, U=U),
        grid=(S // U,),
        in_specs=[
            full((S, B)),
            full((V, E)),
            full((E, 3 * H)),
            full((H, 3 * H)),
            full((1, 3 * H)),
            full((1, 3 * H)),
            full((H, V)),
            full((1, V)),
        ],
        out_specs=pl.BlockSpec((U, B, V), lambda t: (t, 0, 0)),
        out_shape=jax.ShapeDtypeStruct((S, B, V), jnp.float32),
        scratch_shapes=[
            pltpu.VMEM((B, H), jnp.float32),
            pltpu.VMEM((V, 3 * H), jnp.bfloat16),
        ],
    )(x, emb, W_ihT, W_hhT, b_ih2, b_hh2, W_fcT, b_fc2)
    return jnp.swapaxes(out, 0, 1)


# speedup vs baseline: 12.0548x; 1.0177x over previous
import functools

import jax
import jax.numpy as jnp
from jax.experimental import pallas as pl
from jax.experimental.pallas import tpu as pltpu


def _gru_kernel(x_ref, emb_ref, W_ihT_ref, W_hhT_ref, b_ih_ref, b_hh_ref,
                W_fcT_ref, b_fc_ref, out_ref, h_ref, G_ref, *, H, U):
    t = pl.program_id(0)
    B = h_ref.shape[0]
    V = G_ref.shape[0]

    @pl.when(t == 0)
    def _init():
        G_ref[...] = (
            jnp.dot(emb_ref[...], W_ihT_ref[...],
                    preferred_element_type=jnp.float32)
            + b_ih_ref[...]
        ).astype(jnp.bfloat16)
        h_ref[...] = jnp.zeros_like(h_ref)

    h = h_ref[...]
    for u in range(U):
        idx = x_ref[t * U + u]
        onehot = (idx[:, None]
                  == jax.lax.broadcasted_iota(jnp.int32, (B, V), 1)
                  ).astype(jnp.bfloat16)
        gi = jnp.dot(onehot, G_ref[...], preferred_element_type=jnp.float32)

        gh = (jnp.dot(h.astype(jnp.bfloat16), W_hhT_ref[...],
                      preferred_element_type=jnp.float32)
              + b_hh_ref[...])

        i_r, i_z, i_n = gi[:, :H], gi[:, H:2 * H], gi[:, 2 * H:]
        h_r, h_z, h_n = gh[:, :H], gh[:, H:2 * H], gh[:, 2 * H:]
        r = jax.nn.sigmoid(i_r + h_r)
        z = jax.nn.sigmoid(i_z + h_z)
        n = jnp.tanh(i_n + r * h_n)
        h = (1.0 - z) * n + z * h

        out_ref[u] = (
            jnp.dot(h.astype(jnp.bfloat16), W_fcT_ref[...],
                    preferred_element_type=jnp.float32)
            + b_fc_ref[...]
        )
    h_ref[...] = h


def kernel(x_in, emb, W_ih, W_hh, b_ih, b_hh, W_fc, b_fc):
    B, S = x_in.shape
    V, E = emb.shape
    H = W_hh.shape[1]

    x = x_in.astype(jnp.int32).T
    W_ihT = W_ih.T
    W_hhT = W_hh.T.astype(jnp.bfloat16)
    W_fcT = W_fc.T.astype(jnp.bfloat16)
    b_ih2 = b_ih.reshape(1, -1)
    b_hh2 = b_hh.reshape(1, -1)
    b_fc2 = b_fc.reshape(1, -1)

    U = 8
    full = lambda shape: pl.BlockSpec(shape, lambda t: (0,) * len(shape))
    out = pl.pallas_call(
        functools.partial(_gru_kernel, H=H, U=U),
        grid=(S // U,),
        in_specs=[
            full((S, B)),
            full((V, E)),
            full((E, 3 * H)),
            full((H, 3 * H)),
            full((1, 3 * H)),
            full((1, 3 * H)),
            full((H, V)),
            full((1, V)),
        ],
        out_specs=pl.BlockSpec((U, B, V), lambda t: (t, 0, 0)),
        out_shape=jax.ShapeDtypeStruct((S, B, V), jnp.float32),
        scratch_shapes=[
            pltpu.VMEM((B, H), jnp.float32),
            pltpu.VMEM((V, 3 * H), jnp.bfloat16),
        ],
    )(x, emb, W_ihT, W_hhT, b_ih2, b_hh2, W_fcT, b_fc2)
    return jnp.swapaxes(out, 0, 1)


# speedup vs baseline: 12.4585x; 1.0335x over previous
import functools

import jax
import jax.numpy as jnp
from jax.experimental import pallas as pl
from jax.experimental.pallas import tpu as pltpu


def _gru_kernel(x_ref, emb_ref, W_ihT_ref, W_hhT_ref, b_comb_ref, b_hhn_ref,
                W_fcT_ref, b_fc_ref, out_ref, h_ref, G_ref, *, H, U):
    t = pl.program_id(0)
    B = h_ref.shape[0]
    V = G_ref.shape[0]

    @pl.when(t == 0)
    def _init():
        G_ref[...] = (
            jnp.dot(emb_ref[...], W_ihT_ref[...],
                    preferred_element_type=jnp.float32)
            + b_comb_ref[...]
        ).astype(jnp.bfloat16)
        h_ref[...] = jnp.zeros_like(h_ref)

    h = h_ref[...]
    logits = []
    for u in range(U):
        idx = x_ref[t * U + u]
        onehot = (idx[:, None]
                  == jax.lax.broadcasted_iota(jnp.int32, (B, V), 1)
                  ).astype(jnp.bfloat16)
        gi = jnp.dot(onehot, G_ref[...], preferred_element_type=jnp.float32)

        gh = jnp.dot(h.astype(jnp.bfloat16), W_hhT_ref[...],
                     preferred_element_type=jnp.float32)

        r = jax.nn.sigmoid(gi[:, :H] + gh[:, :H])
        z = jax.nn.sigmoid(gi[:, H:2 * H] + gh[:, H:2 * H])
        n = jnp.tanh(gi[:, 2 * H:] + r * (gh[:, 2 * H:] + b_hhn_ref[...]))
        h = (1.0 - z) * n + z * h

        logits.append(
            jnp.dot(h.astype(jnp.bfloat16), W_fcT_ref[...],
                    preferred_element_type=jnp.float32)
            + b_fc_ref[...]
        )
    h_ref[...] = h
    out_ref[...] = jnp.stack(logits, axis=1)


def kernel(x_in, emb, W_ih, W_hh, b_ih, b_hh, W_fc, b_fc):
    B, S = x_in.shape
    V, E = emb.shape
    H = W_hh.shape[1]

    x = x_in.astype(jnp.int32).T
    W_ihT = W_ih.T
    W_hhT = W_hh.astype(jnp.bfloat16).T
    W_fcT = W_fc.astype(jnp.bfloat16).T
    b_comb = (b_ih + jnp.concatenate(
        [b_hh[:2 * H], jnp.zeros_like(b_hh[2 * H:])])).reshape(1, -1)
    b_hhn = b_hh[2 * H:].reshape(1, -1)
    b_fc2 = b_fc.reshape(1, -1)

    U = 8
    full = lambda shape: pl.BlockSpec(shape, lambda t: (0,) * len(shape))
    out = pl.pallas_call(
        functools.partial(_gru_kernel, H=H, U=U),
        grid=(S // U,),
        in_specs=[
            full((S, B)),
            full((V, E)),
            full((E, 3 * H)),
            full((H, 3 * H)),
            full((1, 3 * H)),
            full((1, H)),
            full((H, V)),
            full((1, V)),
        ],
        out_specs=pl.BlockSpec((B, U, V), lambda t: (0, t, 0)),
        out_shape=jax.ShapeDtypeStruct((B, S, V), jnp.float32),
        scratch_shapes=[
            pltpu.VMEM((B, H), jnp.float32),
            pltpu.VMEM((V, 3 * H), jnp.bfloat16),
        ],
    )(x, emb, W_ihT, W_hhT, b_comb, b_hhn, W_fcT, b_fc2)
    return out


# speedup vs baseline: 14.2816x; 1.1463x over previous
import functools

import jax
import jax.numpy as jnp
from jax.experimental import pallas as pl
from jax.experimental.pallas import tpu as pltpu


def _gru_kernel(x_ref, emb_ref, W_ih_ref, W_hh_ref, b_comb_ref, b_hhn_ref,
                W_fc_ref, b_fc_ref, out_ref, h_ref, G_ref, Whh_ref, Wfc_ref,
                *, H, U):
    t = pl.program_id(0)
    B = h_ref.shape[0]
    V = G_ref.shape[0]

    @pl.when(t == 0)
    def _init():
        Whh_ref[...] = W_hh_ref[...].astype(jnp.bfloat16).T
        Wfc_ref[...] = W_fc_ref[...].astype(jnp.bfloat16).T
        G_ref[...] = (
            jax.lax.dot_general(emb_ref[...], W_ih_ref[...],
                                (((1,), (1,)), ((), ())),
                                preferred_element_type=jnp.float32)
            + b_comb_ref[...]
        ).astype(jnp.bfloat16)
        h_ref[...] = jnp.zeros_like(h_ref)

    h = h_ref[...]
    logits = []
    for u in range(U):
        idx = x_ref[t * U + u]
        onehot = (idx[:, None]
                  == jax.lax.broadcasted_iota(jnp.int32, (B, V), 1)
                  ).astype(jnp.bfloat16)
        gi = jnp.dot(onehot, G_ref[...], preferred_element_type=jnp.float32)

        gh = jnp.dot(h.astype(jnp.bfloat16), Whh_ref[...],
                     preferred_element_type=jnp.float32)

        r = jax.nn.sigmoid(gi[:, :H] + gh[:, :H])
        z = jax.nn.sigmoid(gi[:, H:2 * H] + gh[:, H:2 * H])
        n = jnp.tanh(gi[:, 2 * H:] + r * (gh[:, 2 * H:] + b_hhn_ref[...]))
        h = (1.0 - z) * n + z * h

        logits.append(
            jnp.dot(h.astype(jnp.bfloat16), Wfc_ref[...],
                    preferred_element_type=jnp.float32)
            + b_fc_ref[...]
        )
    h_ref[...] = h
    out_ref[...] = jnp.stack(logits, axis=1)


def kernel(x_in, emb, W_ih, W_hh, b_ih, b_hh, W_fc, b_fc):
    B, S = x_in.shape
    V, E = emb.shape
    H = W_hh.shape[1]

    x = x_in.astype(jnp.int32).T
    b_comb = (b_ih + jnp.concatenate(
        [b_hh[:2 * H], jnp.zeros_like(b_hh[2 * H:])])).reshape(1, -1)
    b_hhn = b_hh[2 * H:].reshape(1, -1)
    b_fc2 = b_fc.reshape(1, -1)

    U = 8
    full = lambda shape: pl.BlockSpec(shape, lambda t: (0,) * len(shape))
    out = pl.pallas_call(
        functools.partial(_gru_kernel, H=H, U=U),
        grid=(S // U,),
        in_specs=[
            full((S, B)),
            full((V, E)),
            full((3 * H, E)),
            full((3 * H, H)),
            full((1, 3 * H)),
            full((1, H)),
            full((V, H)),
            full((1, V)),
        ],
        out_specs=pl.BlockSpec((B, U, V), lambda t: (0, t, 0)),
        out_shape=jax.ShapeDtypeStruct((B, S, V), jnp.float32),
        scratch_shapes=[
            pltpu.VMEM((B, H), jnp.float32),
            pltpu.VMEM((V, 3 * H), jnp.bfloat16),
            pltpu.VMEM((H, 3 * H), jnp.bfloat16),
            pltpu.VMEM((H, V), jnp.bfloat16),
        ],
    )(x, emb, W_ih, W_hh, b_comb, b_hhn, W_fc, b_fc2)
    return out
